# Initial kernel scaffold; baseline (speedup 1.0000x reference)
import jax
import jax.numpy as jnp
from jax.experimental import pallas as pl


def kernel(x, edge_index, W1_l, b1_l, W1_r, W2_l, b2_l, W2_r):
    raise NotImplementedError("write your pallas kernel here")


# speedup vs baseline: 6.9840x; 6.9840x over previous
import functools

import jax
import jax.numpy as jnp
from jax import lax
from jax.experimental import pallas as pl
from jax.experimental.pallas import tpu as pltpu
from jax.experimental.pallas import tpu_sc as plsc

N_NODES = 10000
D = 128
E_EDGES = 320000

NC, NS = 2, 16
NW = NC * NS
EPW = E_EDGES // NW
K = 80
C = EPW // K
NP = 10240
ROWS_PER_TILE = NP // NS
CNT_W = 128

_MESH = plsc.VectorSubcoreMesh(core_axis_name="c", subcore_axis_name="s")


@functools.partial(
    pl.kernel,
    out_type=jax.ShapeDtypeStruct((NC, NP, D), jnp.float32),
    mesh=_MESH,
    scratch_types=[
        pltpu.VMEM((C, K), jnp.int32),
        pltpu.VMEM((C, K), jnp.int32),
        pltpu.VMEM((K, D), jnp.float32),
        pltpu.SemaphoreType.DMA,
        pltpu.VMEM_SHARED((NP, D), jnp.float32),
    ],
)
def _sc_agg(x_hbm, src_hbm, dst_hbm, zd_hbm, agg_hbm,
            src_v, dst_v, rows_v, sem, acc):
  c = lax.axis_index("c")
  s = lax.axis_index("s")
  wid = c * NS + s
  r0 = s * ROWS_PER_TILE

  pltpu.sync_copy(src_hbm.at[wid], src_v)
  pltpu.sync_copy(dst_hbm.at[wid], dst_v)
  pltpu.sync_copy(zd_hbm.at[pl.ds(r0, ROWS_PER_TILE)],
                  acc.at[pl.ds(r0, ROWS_PER_TILE)])
  plsc.subcore_barrier()

  def chunk(j, carry):
    pltpu.async_copy(x_hbm.at[src_v.at[j]], rows_v, sem).wait()
    pltpu.sync_copy(rows_v, acc.at[dst_v.at[j]], add=True)
    return carry

  lax.fori_loop(0, C, chunk, 0)
  plsc.subcore_barrier()

  pltpu.sync_copy(acc.at[pl.ds(r0, ROWS_PER_TILE)],
                  agg_hbm.at[c, pl.ds(r0, ROWS_PER_TILE)])


@functools.partial(
    pl.kernel,
    out_type=jax.ShapeDtypeStruct((NC, NP, CNT_W), jnp.float32),
    mesh=_MESH,
    scratch_types=[
        pltpu.VMEM((C, K), jnp.int32),
        pltpu.VMEM((K, CNT_W), jnp.float32),
        pltpu.VMEM_SHARED((NP, CNT_W), jnp.float32),
    ],
)
def _sc_cnt(dst_hbm, zc_hbm, ones_hbm, cnt_hbm, dst_v, ones_v, cacc):
  c = lax.axis_index("c")
  s = lax.axis_index("s")
  wid = c * NS + s
  r0 = s * ROWS_PER_TILE

  pltpu.sync_copy(dst_hbm.at[wid], dst_v)
  pltpu.sync_copy(ones_hbm, ones_v)
  pltpu.sync_copy(zc_hbm.at[pl.ds(r0, ROWS_PER_TILE)],
                  cacc.at[pl.ds(r0, ROWS_PER_TILE)])
  plsc.subcore_barrier()

  def chunk(j, carry):
    pltpu.sync_copy(ones_v, cacc.at[dst_v.at[j]], add=True)
    return carry

  lax.fori_loop(0, C, chunk, 0)
  plsc.subcore_barrier()

  pltpu.sync_copy(cacc.at[pl.ds(r0, ROWS_PER_TILE)],
                  cnt_hbm.at[c, pl.ds(r0, ROWS_PER_TILE)])


def _tc_layer(x, aggp, cntp, W_l, b_l, W_r, relu: bool):
  R = 1000
  grid = (N_NODES // R,)

  def body(x_ref, agg_ref, cnt_ref, wl_ref, wr_ref, b_ref, o_ref):
    agg = agg_ref[0] + agg_ref[1]
    cnt = cnt_ref[0, :, 0:1] + cnt_ref[1, :, 0:1]
    mean = agg / jnp.maximum(cnt, 1.0)
    dn = (((1,), (1,)), ((), ()))
    out = (lax.dot_general(mean, wl_ref[...], dn,
                           preferred_element_type=jnp.float32)
           + lax.dot_general(x_ref[...], wr_ref[...], dn,
                             preferred_element_type=jnp.float32)
           + b_ref[...])
    if relu:
      out = jnp.maximum(out, 0.0)
    o_ref[...] = out

  return pl.pallas_call(
      body,
      grid=grid,
      in_specs=[
          pl.BlockSpec((R, D), lambda i: (i, 0)),
          pl.BlockSpec((NC, R, D), lambda i: (0, i, 0)),
          pl.BlockSpec((NC, R, CNT_W), lambda i: (0, i, 0)),
          pl.BlockSpec((D, D), lambda i: (0, 0)),
          pl.BlockSpec((D, D), lambda i: (0, 0)),
          pl.BlockSpec((1, D), lambda i: (0, 0)),
      ],
      out_specs=pl.BlockSpec((R, D), lambda i: (i, 0)),
      out_shape=jax.ShapeDtypeStruct((N_NODES, D), jnp.float32),
  )(x, aggp, cntp, W_l, W_r, b_l.reshape(1, D))


def kernel(x, edge_index, W1_l, b1_l, W1_r, W2_l, b2_l, W2_r):
  src = edge_index[0].reshape(NW, C, K)
  dst = edge_index[1].reshape(NW, C, K)
  zd = jnp.zeros((NP, D), jnp.float32)
  zc = jnp.zeros((NP, CNT_W), jnp.float32)
  ones = jnp.ones((K, CNT_W), jnp.float32)

  cntp = _sc_cnt(dst, zc, ones)
  agg1p = _sc_agg(x, src, dst, zd)
  h = _tc_layer(x, agg1p, cntp, W1_l, b1_l, W1_r, relu=True)
  agg2p = _sc_agg(h, src, dst, zd)
  return _tc_layer(h, agg2p, cntp, W2_l, b2_l, W2_r, relu=False)
